# Initial kernel scaffold; baseline (speedup 1.0000x reference)
import jax
import jax.numpy as jnp
from jax.experimental import pallas as pl


def kernel(flat, segment_ids):
    raise NotImplementedError("write your pallas kernel here")


# speedup vs baseline: 3.5834x; 3.5834x over previous
import jax
import jax.numpy as jnp
from jax import lax
from jax.experimental import pallas as pl
from jax.experimental.pallas import tpu as pltpu
from jax.experimental.pallas import tpu_sc as plsc

NUM_SEG = 16
TOTAL_TOK = 32768
D = 128

NC = 2
NS = 16
NW = NC * NS
TW = TOTAL_TOK // NW
R = 128
NCHUNK = TW // R


def _sc_body(flat_hbm, sids_hbm, zacc_hbm, psums_hbm, sid_v, rows_v, shared_acc):
    c = lax.axis_index("c")
    s = lax.axis_index("s")
    wid = c * NS + s

    pltpu.sync_copy(sids_hbm.at[wid], sid_v)

    @pl.when(s == 0)
    def _init():
        pltpu.sync_copy(zacc_hbm, shared_acc)

    plsc.subcore_barrier()

    for k in range(NCHUNK):
        base = wid * TW + k * R
        pltpu.sync_copy(flat_hbm.at[pl.ds(base, R)], rows_v)
        pltpu.sync_copy(rows_v, shared_acc.at[sid_v.at[k]], add=True)

    plsc.subcore_barrier()

    @pl.when(s == 0)
    def _fini():
        pltpu.sync_copy(shared_acc, psums_hbm.at[c])


def _combine_body(ps_ref, sid_ref, o_ref):
    sids = sid_ref[...]
    seg = lax.broadcasted_iota(jnp.int32, (NUM_SEG, TOTAL_TOK // D, D), 0)
    eq = (sids[None, :, :] == seg).astype(jnp.float32)
    cnt = jnp.sum(eq, axis=(1, 2))
    cnt = jnp.maximum(cnt, 1.0)[:, None]
    o_ref[...] = (ps_ref[0] + ps_ref[1]) / cnt


def kernel(flat, segment_ids):
    sids = segment_ids.astype(jnp.int32)
    sids3 = sids.reshape(NW, NCHUNK, R)
    zacc = jnp.zeros((NUM_SEG, D), jnp.float32)

    mesh = plsc.VectorSubcoreMesh(core_axis_name="c", subcore_axis_name="s")
    psums = pl.kernel(
        _sc_body,
        out_type=jax.ShapeDtypeStruct((NC, NUM_SEG, D), jnp.float32),
        mesh=mesh,
        scratch_types=[
            pltpu.VMEM((NCHUNK, R), jnp.int32),
            pltpu.VMEM((R, D), jnp.float32),
            pltpu.VMEM_SHARED((NUM_SEG, D), jnp.float32),
        ],
    )(flat, sids3, zacc)

    out = pl.pallas_call(
        _combine_body,
        out_shape=jax.ShapeDtypeStruct((NUM_SEG, D), jnp.float32),
    )(psums, sids.reshape(TOTAL_TOK // D, D))
    return out


# speedup vs baseline: 4.1897x; 1.1692x over previous
import jax
import jax.numpy as jnp
from jax import lax
from jax.experimental import pallas as pl
from jax.experimental.pallas import tpu as pltpu
from jax.experimental.pallas import tpu_sc as plsc

NUM_SEG = 16
TOTAL_TOK = 32768
D = 128

NC = 2
NS = 16
NW = NC * NS
TW = TOTAL_TOK // NW
R = 128
NCHUNK = TW // R


BR = 256
NB = TW // BR


def _sc_body(flat_hbm, sids_hbm, zacc_hbm, psums_hbm,
             sid_v, rows_a, rows_b, shared_acc, sem_a, sem_b):
    c = lax.axis_index("c")
    s = lax.axis_index("s")
    wid = c * NS + s
    base = wid * TW

    bufs = (rows_a, rows_b)
    sems = (sem_a, sem_b)

    pltpu.sync_copy(sids_hbm.at[wid], sid_v)

    @pl.when(s == 0)
    def _init():
        pltpu.sync_copy(zacc_hbm, shared_acc)

    plsc.subcore_barrier()

    pending = pltpu.async_copy(flat_hbm.at[pl.ds(base, BR)], bufs[0], sems[0])
    for k in range(NB):
        nxt = None
        if k + 1 < NB:
            nxt = pltpu.async_copy(
                flat_hbm.at[pl.ds(base + (k + 1) * BR, BR)],
                bufs[(k + 1) % 2], sems[(k + 1) % 2])
        pending.wait()
        cur = bufs[k % 2]
        for h in range(BR // R):
            pltpu.sync_copy(cur.at[pl.ds(h * R, R)],
                            shared_acc.at[sid_v.at[k * (BR // R) + h]],
                            add=True)
        pending = nxt

    plsc.subcore_barrier()

    @pl.when(s == 0)
    def _fini():
        pltpu.sync_copy(shared_acc, psums_hbm.at[c])


def _combine_body(ps_ref, sid_ref, o_ref):
    sids = sid_ref[...]
    seg = lax.broadcasted_iota(jnp.int32, (NUM_SEG, TOTAL_TOK // D, D), 0)
    eq = (sids[None, :, :] == seg).astype(jnp.float32)
    cnt = jnp.sum(eq, axis=(1, 2))
    cnt = jnp.maximum(cnt, 1.0)[:, None]
    o_ref[...] = (ps_ref[0] + ps_ref[1]) / cnt


def kernel(flat, segment_ids):
    sids = segment_ids.astype(jnp.int32)
    sids3 = sids.reshape(NW, NCHUNK, R)
    zacc = jnp.zeros((NUM_SEG, D), jnp.float32)

    mesh = plsc.VectorSubcoreMesh(core_axis_name="c", subcore_axis_name="s")
    psums = pl.kernel(
        _sc_body,
        out_type=jax.ShapeDtypeStruct((NC, NUM_SEG, D), jnp.float32),
        mesh=mesh,
        scratch_types=[
            pltpu.VMEM((NCHUNK, R), jnp.int32),
            pltpu.VMEM((BR, D), jnp.float32),
            pltpu.VMEM((BR, D), jnp.float32),
            pltpu.VMEM_SHARED((NUM_SEG, D), jnp.float32),
            pltpu.SemaphoreType.DMA,
            pltpu.SemaphoreType.DMA,
        ],
    )(flat, sids3, zacc)

    out = pl.pallas_call(
        _combine_body,
        out_shape=jax.ShapeDtypeStruct((NUM_SEG, D), jnp.float32),
    )(psums, sids.reshape(TOTAL_TOK // D, D))
    return out
